# Initial kernel scaffold; baseline (speedup 1.0000x reference)
import jax
import jax.numpy as jnp
from jax.experimental import pallas as pl


def kernel(x, flat, edge_index, W1, b1, W2, b2):
    raise NotImplementedError("write your pallas kernel here")


# speedup vs baseline: 10.7323x; 10.7323x over previous
import functools

import jax
import jax.numpy as jnp
from jax import lax
from jax.experimental import pallas as pl
from jax.experimental.pallas import tpu as pltpu
from jax.experimental.pallas import tpu_sc as plsc

N = 10000
E = 320000
D = 128

NC = 2
NS = 16
NW = NC * NS
CH = 128
NCHUNK = -(-E // (NW * CH))
EPW = NCHUNK * CH
E_PAD = EPW * NW
NP = N + 112
RPT = NP // NS

_mesh = plsc.VectorSubcoreMesh(core_axis_name="c", subcore_axis_name="s")


def _deg_body(dst_hbm, ones_hbm, zeros_hbm, degp_hbm, acc, idx_v, ones_v):
    cid = lax.axis_index("c")
    sid = lax.axis_index("s")
    wid = sid * NC + cid
    row0 = pl.multiple_of(sid * RPT, 8)
    pltpu.sync_copy(zeros_hbm.at[pl.ds(row0, RPT)], acc.at[pl.ds(row0, RPT)])
    pltpu.sync_copy(ones_hbm, ones_v)
    plsc.subcore_barrier()

    def body(j, carry):
        off = pl.multiple_of(wid * EPW + j * CH, 8)
        pltpu.sync_copy(dst_hbm.at[pl.ds(off, CH)], idx_v)
        pltpu.sync_copy(ones_v, acc.at[idx_v], add=True)
        return carry

    lax.fori_loop(0, NCHUNK, body, 0)
    plsc.subcore_barrier()
    pltpu.sync_copy(acc.at[pl.ds(row0, RPT)],
                    degp_hbm.at[cid, pl.ds(row0, RPT)])


_deg_call = pl.kernel(
    _deg_body,
    out_type=jax.ShapeDtypeStruct((NC, NP, 16), jnp.float32),
    mesh=_mesh,
    scratch_types=[
        pltpu.VMEM_SHARED((NP, 16), jnp.float32),
        pltpu.VMEM((CH,), jnp.int32),
        pltpu.VMEM((CH, 16), jnp.float32),
    ],
    compiler_params=pltpu.CompilerParams(use_tc_tiling_on_sc=False),
)


def _edge_body(h_hbm, src_hbm, dst_hbm, zeros_hbm, part_hbm,
               acc, idx_s, idx_d, rows, sem):
    cid = lax.axis_index("c")
    sid = lax.axis_index("s")
    wid = sid * NC + cid
    row0 = pl.multiple_of(sid * RPT, 8)
    pltpu.sync_copy(zeros_hbm.at[pl.ds(row0, RPT)], acc.at[pl.ds(row0, RPT)])
    plsc.subcore_barrier()

    def body(j, carry):
        off = pl.multiple_of(wid * EPW + j * CH, 8)
        pltpu.sync_copy(src_hbm.at[pl.ds(off, CH)], idx_s)
        pltpu.sync_copy(dst_hbm.at[pl.ds(off, CH)], idx_d)
        pltpu.async_copy(h_hbm.at[idx_s], rows, sem).wait()
        pltpu.sync_copy(rows, acc.at[idx_d], add=True)
        return carry

    lax.fori_loop(0, NCHUNK, body, 0)
    plsc.subcore_barrier()
    pltpu.sync_copy(acc.at[pl.ds(row0, RPT)],
                    part_hbm.at[cid, pl.ds(row0, RPT)])


_edge_call = pl.kernel(
    _edge_body,
    out_type=jax.ShapeDtypeStruct((NC, NP, D), jnp.float32),
    mesh=_mesh,
    scratch_types=[
        pltpu.VMEM_SHARED((NP, D), jnp.float32),
        pltpu.VMEM((CH,), jnp.int32),
        pltpu.VMEM((CH,), jnp.int32),
        pltpu.VMEM((CH, D), jnp.float32),
        pltpu.SemaphoreType.DMA,
    ],
)


RB = 1024
GRID = -(-N // RB)


def _dinv_of(degp_ref):
    deg = degp_ref[0, :, 0:1] + degp_ref[1, :, 0:1] + 1.0
    return lax.rsqrt(deg)


def _stage_a_body(x_ref, w_ref, degp_ref, out_ref):
    h = jnp.dot(x_ref[:], w_ref[:], preferred_element_type=jnp.float32)
    out_ref[:] = h * _dinv_of(degp_ref)


def _stage_b_body(p_ref, h1_ref, degp_ref, w_ref, b_ref, out_ref):
    dinv = _dinv_of(degp_ref)
    s = p_ref[0] + p_ref[1] + h1_ref[:]
    z = jnp.maximum(dinv * s + b_ref[:], 0.0)
    out_ref[:] = jnp.dot(z, w_ref[:], preferred_element_type=jnp.float32) * dinv


def _stage_c_body(q_ref, h2_ref, degp_ref, b_ref, out_ref):
    dinv = _dinv_of(degp_ref)
    out_ref[:] = dinv * (q_ref[0] + q_ref[1] + h2_ref[:]) + b_ref[:]


_deg_spec = pl.BlockSpec((NC, RB, 16), lambda i: (0, i, 0))
_row_spec = pl.BlockSpec((RB, D), lambda i: (i, 0))
_part_spec = pl.BlockSpec((NC, RB, D), lambda i: (0, i, 0))
_w_spec = pl.BlockSpec((D, D), lambda i: (0, 0))
_b_spec = pl.BlockSpec((1, D), lambda i: (0, 0))

_stage_a = pl.pallas_call(
    _stage_a_body,
    grid=(GRID,),
    in_specs=[_row_spec, _w_spec, _deg_spec],
    out_specs=_row_spec,
    out_shape=jax.ShapeDtypeStruct((N, D), jnp.float32),
)

_stage_b = pl.pallas_call(
    _stage_b_body,
    grid=(GRID,),
    in_specs=[_part_spec, _row_spec, _deg_spec, _w_spec, _b_spec],
    out_specs=_row_spec,
    out_shape=jax.ShapeDtypeStruct((N, D), jnp.float32),
)

_stage_c = pl.pallas_call(
    _stage_c_body,
    grid=(GRID,),
    in_specs=[_part_spec, _row_spec, _deg_spec, _b_spec],
    out_specs=_row_spec,
    out_shape=jax.ShapeDtypeStruct((N, D), jnp.float32),
)


def kernel(x, flat, edge_index, W1, b1, W2, b2):
    src = edge_index[0]
    dst = edge_index[1]
    pad = E_PAD - E
    srcp = jnp.concatenate([src, jnp.zeros((pad,), jnp.int32)])
    dstp = jnp.concatenate([dst, jnp.full((pad,), N, jnp.int32)])
    ones16 = jnp.ones((CH, 16), jnp.float32)
    zeros16 = jnp.zeros((NP, 16), jnp.float32)
    zeros128 = jnp.zeros((NP, D), jnp.float32)

    degp = _deg_call(dstp, ones16, zeros16)
    h1p = _stage_a(x, W1, degp)
    p = _edge_call(h1p, srcp, dstp, zeros128)
    h2p = _stage_b(p, h1p, degp, W2, b1.reshape(1, D))
    q = _edge_call(h2p, srcp, dstp, zeros128)
    return _stage_c(q, h2p, degp, b2.reshape(1, D))
